# Initial kernel scaffold; baseline (speedup 1.0000x reference)
import functools

import jax
import jax.numpy as jnp
from jax import lax
from jax.experimental import pallas as pl
from jax.experimental.pallas import tpu as pltpu
from jax.experimental.pallas import tpu_sc as plsc

_D = 64
_B = 16384 * 50
_NC = 2
_NS = 16
_NW = _NC * _NS
_CHUNK = 128
_B_PER_W = _B // _NW
_N_CHUNKS = _B_PER_W // _CHUNK

_mesh = plsc.VectorSubcoreMesh(core_axis_name="c", subcore_axis_name="s")


@functools.partial(
    pl.kernel,
    out_type=jax.ShapeDtypeStruct((_B, _D), jnp.float32),
    mesh=_mesh,
    scratch_types=[
        pltpu.VMEM((_CHUNK,), jnp.int32),
        pltpu.VMEM((_CHUNK, _D), jnp.float32),
        pltpu.SemaphoreType.DMA,
    ],
)
def _embed_gather(idx_hbm, table_hbm, out_hbm, idx_v, rows_v, sem):
    wid = lax.axis_index("s") * _NC + lax.axis_index("c")
    base = wid * _B_PER_W

    @pl.loop(0, _N_CHUNKS)
    def _chunk(i):
        off = base + i * _CHUNK
        pltpu.sync_copy(idx_hbm.at[pl.ds(off, _CHUNK)], idx_v)
        pltpu.async_copy(table_hbm.at[idx_v], rows_v, sem).wait()
        pltpu.sync_copy(rows_v, out_hbm.at[pl.ds(off, _CHUNK)])


def kernel(token_ids, embedding_mat):
    idx = token_ids.reshape(-1).astype(jnp.int32)
    out = _embed_gather(idx, embedding_mat)
    return out.reshape(token_ids.shape + (_D,))


# speedup vs baseline: 1.5840x; 1.5840x over previous
import functools

import jax
import jax.numpy as jnp
from jax import lax
from jax.experimental import pallas as pl
from jax.experimental.pallas import tpu as pltpu
from jax.experimental.pallas import tpu_sc as plsc

_D = 64
_B = 16384 * 50
_NC = 2
_NS = 16
_NW = _NC * _NS
_CHUNK = 128
_B_PER_W = _B // _NW
_N_CHUNKS = _B_PER_W // _CHUNK

_mesh = plsc.VectorSubcoreMesh(core_axis_name="c", subcore_axis_name="s")


@functools.partial(
    pl.kernel,
    out_type=jax.ShapeDtypeStruct((_B, _D), jnp.float32),
    mesh=_mesh,
    scratch_types=[
        pltpu.VMEM((_CHUNK,), jnp.int32),
        pltpu.VMEM((_CHUNK, _D), jnp.float32),
        pltpu.SemaphoreType.DMA,
    ],
    compiler_params=pltpu.CompilerParams(use_tc_tiling_on_sc=False),
)
def _embed_gather(idx_hbm, table_hbm, out_hbm, idx_v, rows_v, sem):
    wid = lax.axis_index("s") * _NC + lax.axis_index("c")
    base = wid * _B_PER_W

    @pl.loop(0, _N_CHUNKS)
    def _chunk(i):
        off = base + i * _CHUNK
        pltpu.sync_copy(idx_hbm.at[pl.ds(off, _CHUNK)], idx_v)
        pltpu.async_copy(table_hbm.at[idx_v], rows_v, sem).wait()
        pltpu.sync_copy(rows_v, out_hbm.at[pl.ds(off, _CHUNK)])


def kernel(token_ids, embedding_mat):
    idx = token_ids.reshape(-1).astype(jnp.int32)
    out = _embed_gather(idx, embedding_mat)
    return out.reshape(token_ids.shape + (_D,))


# speedup vs baseline: 1.8375x; 1.1600x over previous
import functools

import jax
import jax.numpy as jnp
from jax import lax
from jax.experimental import pallas as pl
from jax.experimental.pallas import tpu as pltpu
from jax.experimental.pallas import tpu_sc as plsc

_D = 64
_B = 16384 * 50
_NC = 2
_NS = 16
_NW = _NC * _NS
_CHUNK = 128
_NBUF = 4
_B_PER_W = _B // _NW
_N_CHUNKS = _B_PER_W // _CHUNK

_mesh = plsc.VectorSubcoreMesh(core_axis_name="c", subcore_axis_name="s")


@functools.partial(
    pl.kernel,
    out_type=jax.ShapeDtypeStruct((_B, _D), jnp.float32),
    mesh=_mesh,
    scratch_types=[
        pltpu.VMEM((_NBUF, _CHUNK), jnp.int32),
        pltpu.VMEM((_NBUF, _CHUNK, _D), jnp.float32),
        pltpu.SemaphoreType.DMA((_NBUF,)),
        pltpu.SemaphoreType.DMA((_NBUF,)),
        pltpu.SemaphoreType.DMA((_NBUF,)),
    ],
    compiler_params=pltpu.CompilerParams(use_tc_tiling_on_sc=False),
)
def _embed_gather(idx_hbm, table_hbm, out_hbm, idx_v, rows_v,
                  idx_sem, gat_sem, out_sem):
    wid = lax.axis_index("s") * _NC + lax.axis_index("c")
    base = wid * _B_PER_W

    def fire_idx(i, b):
        pltpu.async_copy(idx_hbm.at[pl.ds(base + i * _CHUNK, _CHUNK)],
                         idx_v.at[b], idx_sem.at[b])

    def wait_idx(b):
        pltpu.make_async_copy(idx_hbm.at[pl.ds(base, _CHUNK)],
                              idx_v.at[b], idx_sem.at[b]).wait()

    def fire_gather(b):
        pltpu.async_copy(table_hbm.at[idx_v.at[b]], rows_v.at[b],
                         gat_sem.at[b])

    def wait_gather(b):
        pltpu.make_async_copy(table_hbm.at[idx_v.at[b]], rows_v.at[b],
                              gat_sem.at[b]).wait()

    def fire_out(i, b):
        pltpu.async_copy(rows_v.at[b],
                         out_hbm.at[pl.ds(base + i * _CHUNK, _CHUNK)],
                         out_sem.at[b])

    def wait_out(b):
        pltpu.make_async_copy(rows_v.at[b],
                              out_hbm.at[pl.ds(base, _CHUNK)],
                              out_sem.at[b]).wait()

    for b in range(_NBUF):
        fire_idx(b, b)

    @pl.loop(0, _N_CHUNKS, step=_NBUF)
    def _block(i0):
        @pl.when(i0 > 0)
        def _():
            for b in range(_NBUF):
                wait_out(b)

        for b in range(_NBUF):
            wait_idx(b)
            fire_gather(b)

        for b in range(_NBUF):
            wait_gather(b)
            fire_out(i0 + b, b)

        @pl.when(i0 + _NBUF < _N_CHUNKS)
        def _():
            for b in range(_NBUF):
                fire_idx(i0 + _NBUF + b, b)

    for b in range(_NBUF):
        wait_out(b)


def kernel(token_ids, embedding_mat):
    idx = token_ids.reshape(-1).astype(jnp.int32)
    out = _embed_gather(idx, embedding_mat)
    return out.reshape(token_ids.shape + (_D,))


# speedup vs baseline: 1.8716x; 1.0186x over previous
import functools

import jax
import jax.numpy as jnp
from jax import lax
from jax.experimental import pallas as pl
from jax.experimental.pallas import tpu as pltpu
from jax.experimental.pallas import tpu_sc as plsc

_D = 64
_B = 16384 * 50
_NC = 2
_NS = 16
_NW = _NC * _NS
_CHUNK = 128
_NBUF = 8
_B_PER_W = _B // _NW
_N_CHUNKS = _B_PER_W // _CHUNK

_mesh = plsc.VectorSubcoreMesh(core_axis_name="c", subcore_axis_name="s")


@functools.partial(
    pl.kernel,
    out_type=jax.ShapeDtypeStruct((_B, _D), jnp.float32),
    mesh=_mesh,
    scratch_types=[
        pltpu.VMEM((_NBUF, _CHUNK), jnp.int32),
        pltpu.VMEM((_NBUF, _CHUNK, _D), jnp.float32),
        pltpu.SemaphoreType.DMA((_NBUF,)),
        pltpu.SemaphoreType.DMA((_NBUF,)),
        pltpu.SemaphoreType.DMA((_NBUF,)),
    ],
    compiler_params=pltpu.CompilerParams(use_tc_tiling_on_sc=False),
)
def _embed_gather(idx_hbm, table_hbm, out_hbm, idx_v, rows_v,
                  idx_sem, gat_sem, out_sem):
    wid = lax.axis_index("s") * _NC + lax.axis_index("c")
    base = wid * _B_PER_W

    def fire_idx(i, b):
        pltpu.async_copy(idx_hbm.at[pl.ds(base + i * _CHUNK, _CHUNK)],
                         idx_v.at[b], idx_sem.at[b])

    def wait_idx(b):
        pltpu.make_async_copy(idx_hbm.at[pl.ds(base, _CHUNK)],
                              idx_v.at[b], idx_sem.at[b]).wait()

    def fire_gather(b):
        pltpu.async_copy(table_hbm.at[idx_v.at[b]], rows_v.at[b],
                         gat_sem.at[b])

    def wait_gather(b):
        pltpu.make_async_copy(table_hbm.at[idx_v.at[b]], rows_v.at[b],
                              gat_sem.at[b]).wait()

    def fire_out(i, b):
        pltpu.async_copy(rows_v.at[b],
                         out_hbm.at[pl.ds(base + i * _CHUNK, _CHUNK)],
                         out_sem.at[b])

    def wait_out(b):
        pltpu.make_async_copy(rows_v.at[b],
                              out_hbm.at[pl.ds(base, _CHUNK)],
                              out_sem.at[b]).wait()

    for b in range(_NBUF):
        fire_idx(b, b)

    @pl.loop(0, _N_CHUNKS, step=_NBUF)
    def _block(i0):
        @pl.when(i0 > 0)
        def _():
            for b in range(_NBUF):
                wait_out(b)

        for b in range(_NBUF):
            wait_idx(b)
            fire_gather(b)

        for b in range(_NBUF):
            wait_gather(b)
            fire_out(i0 + b, b)

        @pl.when(i0 + _NBUF < _N_CHUNKS)
        def _():
            for b in range(_NBUF):
                fire_idx(i0 + _NBUF + b, b)

    for b in range(_NBUF):
        wait_out(b)


def kernel(token_ids, embedding_mat):
    idx = token_ids.reshape(-1).astype(jnp.int32)
    out = _embed_gather(idx, embedding_mat)
    return out.reshape(token_ids.shape + (_D,))
